# baseline (speedup 1.0000x reference)
import functools

import numpy as np
import jax
import jax.numpy as jnp
from jax import lax
from jax.experimental import pallas as pl
from jax.experimental.pallas import tpu as pltpu
from jax.experimental.pallas import tpu_sc as plsc

_BATCH = 16384
_NUM_FEATURE = 26
_EMB = 16
_BINS = 50
_TOTAL = 100
_INV = 3

_N = _BATCH * _NUM_FEATURE
_OUT_W = _NUM_FEATURE * _EMB
_LANES = 16

_j = np.arange(_BINS)[:, None]
_c = np.arange(3 * _TOTAL)[None, :]
_MHB = (np.abs(_c - (_j + _TOTAL)) <= _INV).astype(np.float32)


def _table_body(mhb_ref, w_ref, e_ref):
    e_ref[...] = lax.dot_general(
        mhb_ref[...], w_ref[...], (((1,), (1,)), ((), ())),
        preferred_element_type=jnp.float32)


def kernel(x, W):
    table = pl.pallas_call(
        _table_body,
        out_shape=jax.ShapeDtypeStruct((_BINS, _EMB), jnp.float32),
    )(jnp.asarray(_MHB), W)
    tab_flat = jnp.pad(table.reshape(_BINS * _EMB), (0, 1024 - _BINS * _EMB))

    idx = x.astype(jnp.int32).reshape(_N)

    info = plsc.get_sparse_core_info()
    nc, ns = info.num_cores, info.num_subcores
    nw = nc * ns
    rows_w = _BATCH // nw
    rows_chunk = 64
    n_chunks = rows_w // rows_chunk
    lk_chunk = rows_chunk * _NUM_FEATURE
    n_k = lk_chunk // _LANES
    buf_w = 512

    l_arr = np.arange(lk_chunk, dtype=np.int32)
    rows_const = jnp.asarray(l_arr // _NUM_FEATURE)
    colb_const = jnp.asarray((l_arr % _NUM_FEATURE) * _EMB)

    mesh = plsc.VectorSubcoreMesh(core_axis_name="c", subcore_axis_name="s")

    @functools.partial(
        pl.kernel,
        out_type=jax.ShapeDtypeStruct((_BATCH, _OUT_W), jnp.float32),
        mesh=mesh,
        scratch_types=[
            pltpu.VMEM((1024,), jnp.float32),
            pltpu.VMEM((lk_chunk,), jnp.int32),
            pltpu.VMEM((lk_chunk,), jnp.int32),
            pltpu.VMEM((lk_chunk,), jnp.int32),
            pltpu.VMEM((lk_chunk,), jnp.int32),
            pltpu.VMEM((rows_chunk, _OUT_W), jnp.float32),
            pltpu.VMEM((rows_chunk, _OUT_W), jnp.float32),
            pltpu.SemaphoreType.DMA,
            pltpu.SemaphoreType.DMA,
            pltpu.SemaphoreType.DMA,
            pltpu.SemaphoreType.DMA,
        ],
        compiler_params=pltpu.CompilerParams(needs_layout_passes=False),
    )
    def _lookup(tab_hbm, idx_hbm, rows_hbm, colb_hbm, out_hbm,
                tab_v, rows_v, colb_v, idx_v0, idx_v1, o0, o1,
                si0, si1, so0, so1):
        wid = lax.axis_index("s") * nc + lax.axis_index("c")
        base = wid * rows_w * _NUM_FEATURE

        pltpu.sync_copy(tab_hbm, tab_v)
        pltpu.sync_copy(rows_hbm, rows_v)
        pltpu.sync_copy(colb_hbm, colb_v)

        idxs = [idx_v0, idx_v1]
        obufs = [o0, o1]
        sis = [si0, si1]
        sos = [so0, so1]
        ihandles = [None, None]
        ohandles = [None, None]

        pltpu.sync_copy(idx_hbm.at[pl.ds(base, lk_chunk)], idx_v0)

        def compute(buf):
            iv = idxs[buf]
            ob = obufs[buf]

            def body(k, carry):
                vidx = iv[pl.ds(k * _LANES, _LANES)] * _EMB
                vrow = rows_v[pl.ds(k * _LANES, _LANES)]
                vcolb = colb_v[pl.ds(k * _LANES, _LANES)]
                for e in range(_EMB):
                    vals = plsc.load_gather(tab_v, [vidx + e])
                    plsc.store_scatter(ob, [vrow, vcolb + e], vals)
                return carry
            lax.fori_loop(0, n_k, body, 0)

        for c in range(n_chunks):
            buf = c % 2
            nb = (c + 1) % 2
            if c + 1 < n_chunks:
                ihandles[nb] = pltpu.async_copy(
                    idx_hbm.at[pl.ds(base + (c + 1) * lk_chunk, lk_chunk)],
                    idxs[nb], sis[nb])
            if c >= 2:
                ohandles[buf].wait()
            compute(buf)
            ohandles[buf] = pltpu.async_copy(
                obufs[buf],
                out_hbm.at[pl.ds(wid * rows_w + c * rows_chunk, rows_chunk)],
                sos[buf])
            if c + 1 < n_chunks:
                ihandles[nb].wait()
        ohandles[n_chunks % 2].wait()
        ohandles[(n_chunks + 1) % 2].wait()

    return _lookup(tab_flat, idx, rows_const, colb_const)


# speedup vs baseline: 1.5618x; 1.5618x over previous
import functools

import numpy as np
import jax
import jax.numpy as jnp
from jax import lax
from jax.experimental import pallas as pl
from jax.experimental.pallas import tpu as pltpu
from jax.experimental.pallas import tpu_sc as plsc

_BATCH = 16384
_NUM_FEATURE = 26
_EMB = 16
_BINS = 50
_TOTAL = 100
_INV = 3

_N = _BATCH * _NUM_FEATURE
_OUT_W = _NUM_FEATURE * _EMB
_LANES = 16

_j = np.arange(_BINS)[:, None]
_c = np.arange(3 * _TOTAL)[None, :]
_MHB = (np.abs(_c - (_j + _TOTAL)) <= _INV).astype(np.float32)


def _table_body(mhb_ref, w_ref, e_ref):
    e_ref[...] = lax.dot_general(
        mhb_ref[...], w_ref[...], (((1,), (1,)), ((), ())),
        preferred_element_type=jnp.float32)


def kernel(x, W):
    table = pl.pallas_call(
        _table_body,
        out_shape=jax.ShapeDtypeStruct((_BINS, _EMB), jnp.float32),
    )(jnp.asarray(_MHB), W)
    tab_flat = jnp.pad(table.reshape(_BINS * _EMB), (0, 1024 - _BINS * _EMB))

    idx = x.astype(jnp.int32).reshape(_N)

    info = plsc.get_sparse_core_info()
    nc, ns = info.num_cores, info.num_subcores
    nw = nc * ns
    rows_w = _BATCH // nw
    rows_chunk = 64
    n_chunks = rows_w // rows_chunk
    lk_chunk = rows_chunk * _NUM_FEATURE
    n_k = lk_chunk // _LANES
    buf_w = 512

    l_arr = np.arange(lk_chunk, dtype=np.int32)
    rows_const = jnp.asarray(l_arr // _NUM_FEATURE)
    colb_const = jnp.asarray((l_arr % _NUM_FEATURE) * _EMB)

    mesh = plsc.VectorSubcoreMesh(core_axis_name="c", subcore_axis_name="s")

    @functools.partial(
        pl.kernel,
        out_type=jax.ShapeDtypeStruct((_BATCH, _OUT_W), jnp.float32),
        mesh=mesh,
        scratch_types=[
            pltpu.VMEM((1024,), jnp.float32),
            pltpu.VMEM((lk_chunk,), jnp.int32),
            pltpu.VMEM((lk_chunk,), jnp.int32),
            pltpu.VMEM((lk_chunk,), jnp.int32),
            pltpu.VMEM((lk_chunk,), jnp.int32),
            pltpu.VMEM((rows_chunk, _OUT_W), jnp.float32),
            pltpu.VMEM((rows_chunk, _OUT_W), jnp.float32),
            pltpu.SemaphoreType.DMA,
            pltpu.SemaphoreType.DMA,
            pltpu.SemaphoreType.DMA,
            pltpu.SemaphoreType.DMA,
        ],
        compiler_params=pltpu.CompilerParams(needs_layout_passes=False),
    )
    def _lookup(tab_hbm, idx_hbm, rows_hbm, colb_hbm, out_hbm,
                tab_v, rows_v, colb_v, idx_v0, idx_v1, o0, o1,
                si0, si1, so0, so1):
        wid = lax.axis_index("s") * nc + lax.axis_index("c")
        base = wid * rows_w * _NUM_FEATURE

        pltpu.sync_copy(tab_hbm, tab_v)
        pltpu.sync_copy(rows_hbm, rows_v)
        pltpu.sync_copy(colb_hbm, colb_v)

        idxs = [idx_v0, idx_v1]
        obufs = [o0, o1]
        sis = [si0, si1]
        sos = [so0, so1]
        ihandles = [None, None]
        ohandles = [None, None]

        pltpu.sync_copy(idx_hbm.at[pl.ds(base, lk_chunk)], idx_v0)

        def compute(buf):
            iv = idxs[buf]
            ob = obufs[buf]

            def body(k, carry):
                vidx = iv[pl.ds(k * _LANES, _LANES)] * _EMB
                vrow = rows_v[pl.ds(k * _LANES, _LANES)]
                vcolb = colb_v[pl.ds(k * _LANES, _LANES)]
                vals = [plsc.load_gather(tab_v, [vidx + e])
                        for e in range(_EMB)]
                for e in range(_EMB):
                    plsc.store_scatter(ob, [vrow, vcolb + e], vals[e])
                return carry
            lax.fori_loop(0, n_k, body, 0)

        for c in range(n_chunks):
            buf = c % 2
            nb = (c + 1) % 2
            if c + 1 < n_chunks:
                ihandles[nb] = pltpu.async_copy(
                    idx_hbm.at[pl.ds(base + (c + 1) * lk_chunk, lk_chunk)],
                    idxs[nb], sis[nb])
            if c >= 2:
                ohandles[buf].wait()
            compute(buf)
            ohandles[buf] = pltpu.async_copy(
                obufs[buf],
                out_hbm.at[pl.ds(wid * rows_w + c * rows_chunk, rows_chunk)],
                sos[buf])
            if c + 1 < n_chunks:
                ihandles[nb].wait()
        ohandles[n_chunks % 2].wait()
        ohandles[(n_chunks + 1) % 2].wait()

    return _lookup(tab_flat, idx, rows_const, colb_const)
